# baseline (speedup 1.0000x reference)
import functools

import jax
import jax.numpy as jnp
from jax import lax
from jax.experimental import pallas as pl
from jax.experimental.pallas import tpu as pltpu
from jax.experimental.pallas import tpu_sc as plsc

BATCH = 4
TOTAL_SEQ = 2048
D_MODEL = 1024
N_POS = 68


def _vbcast(x):
  return lax.broadcast(x, (16,))


def _sc_gather(table):
  mesh = plsc.VectorSubcoreMesh(core_axis_name="c", subcore_axis_name="s")

  @functools.partial(
      pl.kernel,
      mesh=mesh,
      out_type=jax.ShapeDtypeStruct((BATCH, N_POS, D_MODEL), jnp.float32),
      compiler_params=pltpu.CompilerParams(use_tc_tiling_on_sc=True),
      scratch_types=[
          pltpu.VMEM((16,), jnp.int32),
          pltpu.VMEM((8, D_MODEL), jnp.float32),
          pltpu.VMEM((4, D_MODEL), jnp.float32),
          pltpu.SemaphoreType.DMA,
      ],
  )
  def k(table_hbm, out_hbm, idx_v, rows_v, rows4_v, sem):
    wid = lax.axis_index("s") * 2 + lax.axis_index("c")
    b = wid // 8
    off = (wid % 8) * 8

    t = _vbcast(off) + lax.iota(jnp.int32, 16)
    j = lax.div(t, _vbcast(jnp.int32(17)))
    kk = t - j * _vbcast(jnp.int32(17))
    idx_v[...] = (
        _vbcast(b * TOTAL_SEQ)
        + j * _vbcast(jnp.int32(512))
        + kk * _vbcast(jnp.int32(30))
    )
    pltpu.async_copy(table_hbm.at[idx_v.at[pl.ds(0, 8)]], rows_v, sem).wait()
    pltpu.sync_copy(rows_v, out_hbm.at[b, pl.ds(off, 8)])

    @pl.when(wid < BATCH)
    def _():
      idx_v[...] = (
          _vbcast(wid * TOTAL_SEQ + 3 * 512 + 13 * 30)
          + lax.iota(jnp.int32, 16) * _vbcast(jnp.int32(30))
      )
      pltpu.async_copy(
          table_hbm.at[idx_v.at[pl.ds(0, 4)]], rows4_v, sem
      ).wait()
      pltpu.sync_copy(rows4_v, out_hbm.at[wid, pl.ds(64, 4)])

  return k(table)


def kernel(hidden_states):
  table = hidden_states.reshape(BATCH * TOTAL_SEQ, D_MODEL)
  return _sc_gather(table)


# speedup vs baseline: 1.0342x; 1.0342x over previous
import functools

import jax
import jax.numpy as jnp
from jax import lax
from jax.experimental import pallas as pl
from jax.experimental.pallas import tpu as pltpu
from jax.experimental.pallas import tpu_sc as plsc

BATCH = 4
TOTAL_SEQ = 2048
D_MODEL = 1024
N_POS = 68
ROWS = BATCH * N_POS
CHUNK = 16
N_CHUNKS = ROWS // CHUNK


def _vbcast(x):
  return lax.broadcast(x, (16,))


def _sc_gather(table):
  mesh = plsc.VectorSubcoreMesh(core_axis_name="c", subcore_axis_name="s")

  @functools.partial(
      pl.kernel,
      mesh=mesh,
      out_type=jax.ShapeDtypeStruct((ROWS, D_MODEL), jnp.float32),
      scratch_types=[
          pltpu.VMEM((CHUNK,), jnp.int32),
          pltpu.VMEM((CHUNK, D_MODEL), jnp.float32),
          pltpu.SemaphoreType.DMA,
      ],
  )
  def k(table_hbm, out_hbm, idx_v, rows_v, sem):
    wid = lax.axis_index("s") * 2 + lax.axis_index("c")

    @pl.when(wid < N_CHUNKS)
    def _():
      r = _vbcast(wid * CHUNK) + lax.iota(jnp.int32, 16)
      bj = lax.div(r, _vbcast(jnp.int32(17)))
      kk = r - bj * _vbcast(jnp.int32(17))
      b = lax.div(bj, _vbcast(jnp.int32(4)))
      j = bj - b * _vbcast(jnp.int32(4))
      idx_v[...] = (
          b * _vbcast(jnp.int32(TOTAL_SEQ))
          + j * _vbcast(jnp.int32(512))
          + kk * _vbcast(jnp.int32(30))
      )
      pltpu.async_copy(table_hbm.at[idx_v], rows_v, sem).wait()
      pltpu.sync_copy(rows_v, out_hbm.at[pl.ds(wid * CHUNK, CHUNK)])

  return k(table)


def kernel(hidden_states):
  table = hidden_states.reshape(BATCH * TOTAL_SEQ, D_MODEL)
  out = _sc_gather(table)
  return out.reshape(BATCH, N_POS, D_MODEL)
